# baseline (speedup 1.0000x reference)
import jax
import jax.numpy as jnp
from jax.experimental import pallas as pl
from jax.experimental.pallas import tpu as pltpu

_N = 1_000_000
_C = 100
_B = 2000
_GRID = _N // _B


def _body(yp_ref, yt_ref, w_ref, out_ref, acc_ref):
    i = pl.program_id(0)

    @pl.when(i == 0)
    def _init():
        acc_ref[...] = jnp.zeros_like(acc_ref)

    x = yp_ref[...]
    idx = jax.lax.broadcasted_iota(jnp.int32, (_B, _C), 1)
    m = jnp.max(x, axis=1, keepdims=True)
    pred = jnp.min(jnp.where(x == m, idx, _C), axis=1, keepdims=True)
    yt = yt_ref[...]
    correct = pred == yt

    cp = jnp.sum(
        jnp.where((idx == pred) & correct, 1.0, 0.0), axis=0, keepdims=True
    )
    ct = jnp.sum(jnp.where(idx == yt, 1.0, 0.0), axis=0, keepdims=True)
    acc_ref[0:1, :] += cp
    acc_ref[1:2, :] += ct

    @pl.when(i == _GRID - 1)
    def _fin():
        cpf = acc_ref[0:1, :]
        ctf = acc_ref[1:2, :]
        accuracy = jnp.where(ctf > 0, cpf / jnp.maximum(ctf, 1.0), 0.0)
        w = w_ref[...]
        val = jnp.sum(accuracy * w) / jnp.sum(w)
        out_ref[...] = jnp.broadcast_to(val, (1, 1))


def kernel(y_pred, y_true, weights):
    yt2 = y_true.astype(jnp.int32).reshape(_N, 1)
    w2 = weights.reshape(1, _C)
    out = pl.pallas_call(
        _body,
        grid=(_GRID,),
        in_specs=[
            pl.BlockSpec((_B, _C), lambda i: (i, 0)),
            pl.BlockSpec((_B, 1), lambda i: (i, 0)),
            pl.BlockSpec((1, _C), lambda i: (0, 0)),
        ],
        out_specs=pl.BlockSpec((1, 1), lambda i: (0, 0)),
        out_shape=jax.ShapeDtypeStruct((1, 1), jnp.float32),
        scratch_shapes=[pltpu.VMEM((2, _C), jnp.float32)],
    )(y_pred, yt2, w2)
    return out.reshape(())


# speedup vs baseline: 1.5982x; 1.5982x over previous
import functools

import jax
import jax.numpy as jnp
from jax import lax
from jax.experimental import pallas as pl
from jax.experimental.pallas import tpu as pltpu
from jax.experimental.pallas import tpu_sc as plsc

_N = 1_000_000
_C = 100
_B = 2048
_GRID = -(-_N // _B)

_NW = 32
_CHUNK = 31248
_TAIL = _N - (_NW - 1) * _CHUNK
_STEPS = _CHUNK // 16
_TSTEPS = _TAIL // 16
_HB = 128
_HSIZE = 2 * 16 * _HB


def _amax_body(yp_ref, out_ref):
    x = yp_ref[...]
    pred = jnp.argmax(x, axis=1).astype(jnp.int32)
    out_ref[...] = pred


def _sc_hist_body(pred_hbm, yt_hbm, out_hbm, pred_v, yt_v, hist_v):
    wid = lax.axis_index("s") * 2 + lax.axis_index("c")
    base = pl.multiple_of(wid * _CHUNK, 16)

    def _zero(j, _):
        hist_v[pl.ds(j * 16, 16)] = jnp.zeros((16,), jnp.int32)
        return 0

    lax.fori_loop(0, _HSIZE // 16, _zero, 0)

    pltpu.sync_copy(pred_hbm.at[pl.ds(base, _CHUNK)], pred_v.at[pl.ds(0, _CHUNK)])
    pltpu.sync_copy(yt_hbm.at[pl.ds(base, _CHUNK)], yt_v.at[pl.ds(0, _CHUNK)])

    @pl.when(wid == _NW - 1)
    def _tail_copy():
        off = _N - (_TAIL - _CHUNK)
        pltpu.sync_copy(
            pred_hbm.at[pl.ds(off, _TAIL - _CHUNK)],
            pred_v.at[pl.ds(_CHUNK, _TAIL - _CHUNK)],
        )
        pltpu.sync_copy(
            yt_hbm.at[pl.ds(off, _TAIL - _CHUNK)],
            yt_v.at[pl.ds(_CHUNK, _TAIL - _CHUNK)],
        )

    lanes = lax.iota(jnp.int32, 16) * _HB
    ones = jnp.ones((16,), jnp.int32)

    def _step(i, _):
        p = pred_v[pl.ds(i * 16, 16)]
        t = yt_v[pl.ds(i * 16, 16)]
        hit = jnp.where(p == t, 1, 0).astype(jnp.int32)
        plsc.addupdate_scatter(hist_v, [lanes + p], hit)
        plsc.addupdate_scatter(hist_v, [(16 * _HB) + lanes + t], ones)
        return 0

    lax.fori_loop(0, _STEPS, _step, 0)

    @pl.when(wid == _NW - 1)
    def _tail_steps():
        lax.fori_loop(_STEPS, _TSTEPS, _step, 0)

    pltpu.sync_copy(hist_v, out_hbm.at[wid])


def _fin_body(h_ref, w_ref, out_ref):
    h = h_ref[...]
    s = jnp.sum(h, axis=0, keepdims=True)
    cp = jnp.zeros((1, _HB), jnp.int32)
    ct = jnp.zeros((1, _HB), jnp.int32)
    for l in range(16):
        cp = cp + s[0:1, l * _HB : (l + 1) * _HB]
        ct = ct + s[0:1, 16 * _HB + l * _HB : 16 * _HB + (l + 1) * _HB]
    lane = lax.broadcasted_iota(jnp.int32, (1, _HB), 1)
    valid = (lane < _C) & (ct > 0)
    acc = jnp.where(
        valid, cp.astype(jnp.float32) / jnp.maximum(ct, 1).astype(jnp.float32), 0.0
    )
    w = w_ref[...]
    val = jnp.sum(acc * w) / jnp.sum(w)
    out_ref[...] = jnp.broadcast_to(val, (1, 1))


def kernel(y_pred, y_true, weights):
    pred = pl.pallas_call(
        _amax_body,
        grid=(_GRID,),
        in_specs=[pl.BlockSpec((_B, _C), lambda i: (i, 0))],
        out_specs=pl.BlockSpec((_B,), lambda i: (i,)),
        out_shape=jax.ShapeDtypeStruct((_N,), jnp.int32),
    )(y_pred)

    yt32 = y_true.astype(jnp.int32)

    sc_hist = functools.partial(
        pl.kernel,
        mesh=plsc.VectorSubcoreMesh(core_axis_name="c", subcore_axis_name="s"),
        out_type=jax.ShapeDtypeStruct((_NW, _HSIZE), jnp.int32),
        scratch_types=[
            pltpu.VMEM((_TAIL,), jnp.int32),
            pltpu.VMEM((_TAIL,), jnp.int32),
            pltpu.VMEM((_HSIZE,), jnp.int32),
        ],
        compiler_params=pltpu.CompilerParams(needs_layout_passes=False),
    )(_sc_hist_body)
    hists = sc_hist(pred, yt32)

    w2 = jnp.zeros((1, _HB), jnp.float32).at[0, :_C].set(weights)
    out = pl.pallas_call(
        _fin_body,
        in_specs=[
            pl.BlockSpec((_NW, _HSIZE), lambda: (0, 0)),
            pl.BlockSpec((1, _HB), lambda: (0, 0)),
        ],
        out_specs=pl.BlockSpec((1, 1), lambda: (0, 0)),
        out_shape=jax.ShapeDtypeStruct((1, 1), jnp.float32),
    )(hists, w2)
    return out.reshape(())


# speedup vs baseline: 1.8420x; 1.1525x over previous
import functools

import jax
import jax.numpy as jnp
from jax import lax
from jax.experimental import pallas as pl
from jax.experimental.pallas import tpu as pltpu
from jax.experimental.pallas import tpu_sc as plsc

_N = 1_000_000
_C = 100
_B = 2048
_GRID = -(-_N // _B)
_NP = _GRID * _B

_NW = 32
_CHUNK = 31248
_TAIL = _N - (_NW - 1) * _CHUNK
_STEPS = _CHUNK // 16
_TSTEPS = _TAIL // 16
_HB = 128
_HSIZE = 2 * 16 * _HB


def _amax_body(yp_ref, yt_ref, out_ref):
    x = yp_ref[...]
    xt = jnp.swapaxes(x, 0, 1)
    ytv = yt_ref[0]
    idxs = lax.broadcasted_iota(jnp.int32, (_C, _B), 0)
    neg = jnp.float32(-jnp.inf)
    xv = jnp.max(jnp.where(idxs == ytv, xt, neg), axis=0, keepdims=True)
    m = jnp.max(xt, axis=0, keepdims=True)
    correct = xv >= m
    out_ref[...] = jnp.where(correct, ytv, _C).astype(jnp.int32)[0]


def _sc_hist_body(bin_hbm, yt_hbm, out_hbm, bin_v, yt_v, hist_v):
    wid = lax.axis_index("s") * 2 + lax.axis_index("c")
    base = pl.multiple_of(wid * _CHUNK, 16)

    def _zero(j, _):
        hist_v[pl.ds(j * 16, 16)] = jnp.zeros((16,), jnp.int32)
        return 0

    lax.fori_loop(0, _HSIZE // 16, _zero, 0)

    pltpu.sync_copy(bin_hbm.at[pl.ds(base, _CHUNK)], bin_v.at[pl.ds(0, _CHUNK)])
    pltpu.sync_copy(yt_hbm.at[pl.ds(base, _CHUNK)], yt_v.at[pl.ds(0, _CHUNK)])

    @pl.when(wid == _NW - 1)
    def _tail_copy():
        off = _N - (_TAIL - _CHUNK)
        pltpu.sync_copy(
            bin_hbm.at[pl.ds(off, _TAIL - _CHUNK)],
            bin_v.at[pl.ds(_CHUNK, _TAIL - _CHUNK)],
        )
        pltpu.sync_copy(
            yt_hbm.at[pl.ds(off, _TAIL - _CHUNK)],
            yt_v.at[pl.ds(_CHUNK, _TAIL - _CHUNK)],
        )

    lanes = lax.iota(jnp.int32, 16) * _HB
    ones = jnp.ones((16,), jnp.int32)

    def _step(i, _):
        b = bin_v[pl.ds(i * 16, 16)]
        t = yt_v[pl.ds(i * 16, 16)]
        plsc.addupdate_scatter(hist_v, [lanes + b], ones)
        plsc.addupdate_scatter(hist_v, [(16 * _HB) + lanes + t], ones)
        return 0

    lax.fori_loop(0, _STEPS, _step, 0)

    @pl.when(wid == _NW - 1)
    def _tail_steps():
        lax.fori_loop(_STEPS, _TSTEPS, _step, 0)

    pltpu.sync_copy(hist_v, out_hbm.at[wid])


def _fin_body(h_ref, w_ref, out_ref):
    h = h_ref[...]
    s = jnp.sum(h, axis=0, keepdims=True)
    cp = jnp.zeros((1, _HB), jnp.int32)
    ct = jnp.zeros((1, _HB), jnp.int32)
    for l in range(16):
        cp = cp + s[0:1, l * _HB : (l + 1) * _HB]
        ct = ct + s[0:1, 16 * _HB + l * _HB : 16 * _HB + (l + 1) * _HB]
    lane = lax.broadcasted_iota(jnp.int32, (1, _HB), 1)
    valid = (lane < _C) & (ct > 0)
    acc = jnp.where(
        valid, cp.astype(jnp.float32) / jnp.maximum(ct, 1).astype(jnp.float32), 0.0
    )
    w = w_ref[...]
    val = jnp.sum(acc * w) / jnp.sum(w)
    out_ref[...] = jnp.broadcast_to(val, (1, 1))


def kernel(y_pred, y_true, weights):
    yt32 = y_true.astype(jnp.int32)
    ytp = jnp.concatenate([yt32, jnp.zeros((_NP - _N,), jnp.int32)])
    yt3 = ytp.reshape(_GRID, 1, _B)

    masked_bin = pl.pallas_call(
        _amax_body,
        grid=(_GRID,),
        in_specs=[
            pl.BlockSpec((_B, _C), lambda i: (i, 0)),
            pl.BlockSpec((1, 1, _B), lambda i: (i, 0, 0)),
        ],
        out_specs=pl.BlockSpec((_B,), lambda i: (i,)),
        out_shape=jax.ShapeDtypeStruct((_NP,), jnp.int32),
    )(y_pred, yt3)

    sc_hist = functools.partial(
        pl.kernel,
        mesh=plsc.VectorSubcoreMesh(core_axis_name="c", subcore_axis_name="s"),
        out_type=jax.ShapeDtypeStruct((_NW, _HSIZE), jnp.int32),
        scratch_types=[
            pltpu.VMEM((_TAIL,), jnp.int32),
            pltpu.VMEM((_TAIL,), jnp.int32),
            pltpu.VMEM((_HSIZE,), jnp.int32),
        ],
        compiler_params=pltpu.CompilerParams(needs_layout_passes=False),
    )(_sc_hist_body)
    hists = sc_hist(masked_bin, yt32)

    w2 = jnp.zeros((1, _HB), jnp.float32).at[0, :_C].set(weights)
    out = pl.pallas_call(
        _fin_body,
        in_specs=[
            pl.BlockSpec((_NW, _HSIZE), lambda: (0, 0)),
            pl.BlockSpec((1, _HB), lambda: (0, 0)),
        ],
        out_specs=pl.BlockSpec((1, 1), lambda: (0, 0)),
        out_shape=jax.ShapeDtypeStruct((1, 1), jnp.float32),
    )(hists, w2)
    return out.reshape(())
